# baseline (speedup 1.0000x reference)
import functools

import jax
import jax.numpy as jnp
from jax import lax
from jax.experimental import pallas as pl
from jax.experimental.pallas import tpu as pltpu
from jax.experimental.pallas import tpu_sc as plsc


def _sc_geometry():
    try:
        info = plsc.get_sparse_core_info()
        return info.num_cores, info.num_subcores, info.num_lanes
    except Exception:
        return 2, 16, 16


@functools.lru_cache(maxsize=None)
def _build(batch, dom0, dom1):
    num_cores, num_subcores, lanes = _sc_geometry()
    num_workers = num_cores * num_subcores
    assert batch % num_workers == 0
    bpw = batch // num_workers
    chunk = 128
    assert bpw % chunk == 0
    nchunks = bpw // chunk
    assert chunk % lanes == 0

    mesh = plsc.VectorSubcoreMesh(core_axis_name="c", subcore_axis_name="s")

    @functools.partial(
        pl.kernel,
        mesh=mesh,
        out_type=jax.ShapeDtypeStruct((batch,), jnp.float32),
        scratch_types=[
            pltpu.VMEM((2, bpw), jnp.int32),
            pltpu.VMEM((nchunks, chunk), jnp.int32),
            pltpu.VMEM((bpw,), jnp.float32),
            pltpu.SemaphoreType.DMA,
        ],
    )
    def gather_kernel(xt_hbm, w_hbm, out_hbm, xv, idx_v, out_v, sem):
        wid = lax.axis_index("s") * num_cores + lax.axis_index("c")
        base = wid * bpw
        pltpu.sync_copy(xt_hbm.at[:, pl.ds(base, bpw)], xv)
        for t in range(bpw // lanes):
            sl = pl.ds(t * lanes, lanes)
            row = (t * lanes) // chunk
            col = (t * lanes) % chunk
            idx_v[row, pl.ds(col, lanes)] = xv[0, sl] * dom1 + xv[1, sl]
        copies = [
            pltpu.async_copy(
                w_hbm.at[idx_v.at[j]],
                out_v.at[pl.ds(j * chunk, chunk)],
                sem,
            )
            for j in range(nchunks)
        ]
        for c in copies:
            c.wait()
        pltpu.sync_copy(out_v, out_hbm.at[pl.ds(base, bpw)])

    return gather_kernel


def kernel(x, weights):
    batch = x.shape[0]
    dom0, dom1 = weights.shape
    w_flat = weights.reshape(dom0 * dom1)
    xt = x.T
    return _build(batch, dom0, dom1)(xt, w_flat)
